# baseline (speedup 1.0000x reference)
import functools

import jax
import jax.numpy as jnp
from jax.experimental import pallas as pl

_H = 20
_O = 10
_SELF = 6
_IN = 13
_BATCH = 1024
_BB = 512


def _dot(a, b):
    return jnp.dot(a, b, preferred_element_type=jnp.float32,
                   precision=jax.lax.Precision.HIGHEST)


def _fused_body(s_ref,
                rW1, rb1, rW2, rb2,
                hW1, hb1, hW2, hb2,
                oW1, ob1, oW2, ob2,
                C1, B1, C2, B2,
                V1, c1, V2, c2, V3, c3,
                out_ref):
    bb = s_ref.shape[0]
    s = s_ref[...]

    xr_in = s[:, 0, :_SELF]
    er = jnp.maximum(_dot(xr_in, rW1[...]) + rb1[...], 0.0)
    er = jnp.maximum(_dot(er, rW2[...]) + rb2[...], 0.0)

    h_in = s[:, :_H, _SELF:].reshape(bb * _H, _IN - _SELF)
    eh = jnp.maximum(_dot(h_in, hW1[...]) + hb1[...], 0.0)
    eh = jnp.maximum(_dot(eh, hW2[...]) + hb2[...], 0.0)

    o_in = s[:, _H:, _SELF:].reshape(bb * _O, _IN - _SELF)
    eo = jnp.maximum(_dot(o_in, oW1[...]) + ob1[...], 0.0)
    eo = jnp.maximum(_dot(eo, oW2[...]) + ob2[...], 0.0)

    eh3 = eh.reshape(bb, _H, 32)
    eo3 = eo.reshape(bb, _O, 32)
    sh = jnp.sum(eh3, axis=1)
    so = jnp.sum(eo3, axis=1)

    c = C1[...]
    bia = B1[...]
    hr = jnp.maximum(_dot(sh, c[0]) + _dot(so, c[1]) + _dot(er, c[2])
                     + bia[0:1], 0.0)
    uh = _dot(er, c[3]) + _dot(sh, c[4]) + _dot(so, c[5]) + bia[1:2]
    hh = jnp.maximum(_dot(eh, c[6]).reshape(bb, _H, 32) + uh[:, None, :], 0.0)
    uo = _dot(er, c[7]) + _dot(sh, c[8]) + _dot(so, c[9]) + bia[2:3]
    ho = jnp.maximum(_dot(eo, c[10]).reshape(bb, _O, 32) + uo[:, None, :], 0.0)

    sh2 = jnp.sum(hh, axis=1)
    so2 = jnp.sum(ho, axis=1)

    c2m = C2[...]
    hr2 = jnp.maximum(_dot(sh2, c2m[0]) + _dot(so2, c2m[1]) + _dot(hr, c2m[2])
                      + B2[...], 0.0)

    v = jnp.maximum(_dot(hr2, V1[...]) + c1[...], 0.0)
    v = jnp.maximum(_dot(v, V2[...]) + c2[...], 0.0)
    v = _dot(v, V3[...]) + c3[...]
    out_ref[...] = v


def _prep(params):
    def lin(layer):
        W, b = layer
        return W.T, b[None, :]

    rW1, rb1 = lin(params['w_r'][0])
    rW2, rb2 = lin(params['w_r'][1])
    hW1, hb1 = lin(params['w_h'][0])
    hW2, hb2 = lin(params['w_h'][1])
    oW1, ob1 = lin(params['w_o'][0])
    oW2, ob2 = lin(params['w_o'][1])

    def conv_combine(C):
        Ar_h = C['h2r']['W_rel'].T
        Ar_o = C['o2r']['W_rel'].T
        Ar_r = (C['h2r']['W_root'] + C['o2r']['W_root']).T
        br = C['h2r']['b_rel'] + C['o2r']['b_rel']
        Ah_r = C['r2h']['W_rel'].T
        Ah_sh = C['h2h']['W_rel'].T
        Ah_so = C['o2h']['W_rel'].T
        Ah_self = ((C['r2h']['W_root'] + C['o2h']['W_root']
                    + C['h2h']['W_root']).T - C['h2h']['W_rel'].T)
        bh = C['r2h']['b_rel'] + C['o2h']['b_rel'] + C['h2h']['b_rel']
        Ao_r = C['r2o']['W_rel'].T
        Ao_sh = C['h2o']['W_rel'].T
        Ao_so = C['o2o']['W_rel'].T
        Ao_self = ((C['r2o']['W_root'] + C['h2o']['W_root']
                    + C['o2o']['W_root']).T - C['o2o']['W_rel'].T)
        bo = C['r2o']['b_rel'] + C['h2o']['b_rel'] + C['o2o']['b_rel']
        mats = jnp.stack([Ar_h, Ar_o, Ar_r, Ah_r, Ah_sh, Ah_so, Ah_self,
                          Ao_r, Ao_sh, Ao_so, Ao_self])
        bias = jnp.stack([br, bh, bo])
        return mats, bias

    C1m, B1m = conv_combine(params['conv1'])
    C2all, B2all = conv_combine(params['conv2'])
    C2m = C2all[:3]
    B2m = B2all[0:1]

    V1, c1 = lin(params['value'][0])
    V2, c2 = lin(params['value'][1])
    V3, c3 = lin(params['value'][2])

    return (rW1, rb1, rW2, rb2, hW1, hb1, hW2, hb2, oW1, ob1, oW2, ob2,
            C1m, B1m, C2m, B2m, V1, c1, V2, c2, V3, c3)


@functools.partial(jax.jit, static_argnames=('interpret',))
def _run(state_input, weights, interpret=False):
    n_blocks = _BATCH // _BB

    def full(w):
        return pl.BlockSpec(w.shape, lambda i: (0,) * w.ndim)

    in_specs = [pl.BlockSpec((_BB, _H + _O, _IN), lambda i: (i, 0, 0))]
    in_specs += [full(w) for w in weights]
    out_spec = pl.BlockSpec((_BB, 1), lambda i: (i, 0))

    return pl.pallas_call(
        _fused_body,
        grid=(n_blocks,),
        in_specs=in_specs,
        out_specs=out_spec,
        out_shape=jax.ShapeDtypeStruct((_BATCH, 1), jnp.float32),
        interpret=interpret,
    )(state_input, *weights)


def kernel(state_input, params, dropout):
    weights = _prep(params)
    return _run(state_input, weights)


# speedup vs baseline: 1.2605x; 1.2605x over previous
import functools

import jax
import jax.numpy as jnp
from jax.experimental import pallas as pl

_H = 20
_O = 10
_SELF = 6
_IN = 13
_BATCH = 1024
_BB = 128


def _dot(a, b):
    return jnp.dot(a, b, preferred_element_type=jnp.float32,
                   precision=jax.lax.Precision.HIGHEST)


def _fused_body(s_ref,
                rW1, rb1, rW2, rb2,
                hW1, hb1, hW2, hb2,
                oW1, ob1, oW2, ob2,
                C1, B1, C2, B2,
                V1, c1, V2, c2, V3, c3,
                out_ref):
    bb = s_ref.shape[0]
    s = s_ref[...]

    xr_in = s[:, 0, :_SELF]
    er = jnp.maximum(_dot(xr_in, rW1[...]) + rb1[...], 0.0)
    er = jnp.maximum(_dot(er, rW2[...]) + rb2[...], 0.0)

    h_in = s[:, :_H, _SELF:].reshape(bb * _H, _IN - _SELF)
    eh = jnp.maximum(_dot(h_in, hW1[...]) + hb1[...], 0.0)
    eh = jnp.maximum(_dot(eh, hW2[...]) + hb2[...], 0.0)

    o_in = s[:, _H:, _SELF:].reshape(bb * _O, _IN - _SELF)
    eo = jnp.maximum(_dot(o_in, oW1[...]) + ob1[...], 0.0)
    eo = jnp.maximum(_dot(eo, oW2[...]) + ob2[...], 0.0)

    eh3 = eh.reshape(bb, _H, 32)
    eo3 = eo.reshape(bb, _O, 32)
    sh = jnp.sum(eh3, axis=1)
    so = jnp.sum(eo3, axis=1)

    c = C1[...]
    bia = B1[...]
    hr = jnp.maximum(_dot(sh, c[0]) + _dot(so, c[1]) + _dot(er, c[2])
                     + bia[0:1], 0.0)
    uh = _dot(er, c[3]) + _dot(sh, c[4]) + _dot(so, c[5]) + bia[1:2]
    hh = jnp.maximum(_dot(eh, c[6]).reshape(bb, _H, 32) + uh[:, None, :], 0.0)
    uo = _dot(er, c[7]) + _dot(sh, c[8]) + _dot(so, c[9]) + bia[2:3]
    ho = jnp.maximum(_dot(eo, c[10]).reshape(bb, _O, 32) + uo[:, None, :], 0.0)

    sh2 = jnp.sum(hh, axis=1)
    so2 = jnp.sum(ho, axis=1)

    c2m = C2[...]
    hr2 = jnp.maximum(_dot(sh2, c2m[0]) + _dot(so2, c2m[1]) + _dot(hr, c2m[2])
                      + B2[...], 0.0)

    v = jnp.maximum(_dot(hr2, V1[...]) + c1[...], 0.0)
    v = jnp.maximum(_dot(v, V2[...]) + c2[...], 0.0)
    v = _dot(v, V3[...]) + c3[...]
    out_ref[...] = v


def _prep(params):
    def lin(layer):
        W, b = layer
        return W.T, b[None, :]

    rW1, rb1 = lin(params['w_r'][0])
    rW2, rb2 = lin(params['w_r'][1])
    hW1, hb1 = lin(params['w_h'][0])
    hW2, hb2 = lin(params['w_h'][1])
    oW1, ob1 = lin(params['w_o'][0])
    oW2, ob2 = lin(params['w_o'][1])

    def conv_combine(C):
        Ar_h = C['h2r']['W_rel'].T
        Ar_o = C['o2r']['W_rel'].T
        Ar_r = (C['h2r']['W_root'] + C['o2r']['W_root']).T
        br = C['h2r']['b_rel'] + C['o2r']['b_rel']
        Ah_r = C['r2h']['W_rel'].T
        Ah_sh = C['h2h']['W_rel'].T
        Ah_so = C['o2h']['W_rel'].T
        Ah_self = ((C['r2h']['W_root'] + C['o2h']['W_root']
                    + C['h2h']['W_root']).T - C['h2h']['W_rel'].T)
        bh = C['r2h']['b_rel'] + C['o2h']['b_rel'] + C['h2h']['b_rel']
        Ao_r = C['r2o']['W_rel'].T
        Ao_sh = C['h2o']['W_rel'].T
        Ao_so = C['o2o']['W_rel'].T
        Ao_self = ((C['r2o']['W_root'] + C['h2o']['W_root']
                    + C['o2o']['W_root']).T - C['o2o']['W_rel'].T)
        bo = C['r2o']['b_rel'] + C['h2o']['b_rel'] + C['o2o']['b_rel']
        mats = jnp.stack([Ar_h, Ar_o, Ar_r, Ah_r, Ah_sh, Ah_so, Ah_self,
                          Ao_r, Ao_sh, Ao_so, Ao_self])
        bias = jnp.stack([br, bh, bo])
        return mats, bias

    C1m, B1m = conv_combine(params['conv1'])
    C2all, B2all = conv_combine(params['conv2'])
    C2m = C2all[:3]
    B2m = B2all[0:1]

    V1, c1 = lin(params['value'][0])
    V2, c2 = lin(params['value'][1])
    V3, c3 = lin(params['value'][2])

    return (rW1, rb1, rW2, rb2, hW1, hb1, hW2, hb2, oW1, ob1, oW2, ob2,
            C1m, B1m, C2m, B2m, V1, c1, V2, c2, V3, c3)


@functools.partial(jax.jit, static_argnames=('interpret',))
def _run(state_input, weights, interpret=False):
    n_blocks = _BATCH // _BB

    def full(w):
        return pl.BlockSpec(w.shape, lambda i: (0,) * w.ndim)

    in_specs = [pl.BlockSpec((_BB, _H + _O, _IN), lambda i: (i, 0, 0))]
    in_specs += [full(w) for w in weights]
    out_spec = pl.BlockSpec((_BB, 1), lambda i: (i, 0))

    return pl.pallas_call(
        _fused_body,
        grid=(n_blocks,),
        in_specs=in_specs,
        out_specs=out_spec,
        out_shape=jax.ShapeDtypeStruct((_BATCH, 1), jnp.float32),
        interpret=interpret,
    )(state_input, *weights)


def kernel(state_input, params, dropout):
    weights = _prep(params)
    return _run(state_input, weights)


# speedup vs baseline: 1.2913x; 1.0244x over previous
import functools

import jax
import jax.numpy as jnp
from jax.experimental import pallas as pl

_H = 20
_O = 10
_SELF = 6
_IN = 13
_BATCH = 1024
_BB = 256


def _dot(a, b):
    return jnp.dot(a, b, preferred_element_type=jnp.float32,
                   precision=jax.lax.Precision.HIGHEST)


def _fused_body(s_ref,
                rW1, rb1, rW2, rb2,
                hW1, hb1, hW2, hb2,
                oW1, ob1, oW2, ob2,
                C1, B1, C2, B2,
                V1, c1, V2, c2, V3, c3,
                out_ref):
    bb = s_ref.shape[0]
    s = s_ref[...]

    xr_in = s[:, 0, :_SELF]
    er = jnp.maximum(_dot(xr_in, rW1[...]) + rb1[...], 0.0)
    er = jnp.maximum(_dot(er, rW2[...]) + rb2[...], 0.0)

    h_in = s[:, :_H, _SELF:].reshape(bb * _H, _IN - _SELF)
    eh = jnp.maximum(_dot(h_in, hW1[...]) + hb1[...], 0.0)
    eh = jnp.maximum(_dot(eh, hW2[...]) + hb2[...], 0.0)

    o_in = s[:, _H:, _SELF:].reshape(bb * _O, _IN - _SELF)
    eo = jnp.maximum(_dot(o_in, oW1[...]) + ob1[...], 0.0)
    eo = jnp.maximum(_dot(eo, oW2[...]) + ob2[...], 0.0)

    eh3 = eh.reshape(bb, _H, 32)
    eo3 = eo.reshape(bb, _O, 32)
    sh = jnp.sum(eh3, axis=1)
    so = jnp.sum(eo3, axis=1)

    c = C1[...]
    bia = B1[...]
    hr = jnp.maximum(_dot(sh, c[0]) + _dot(so, c[1]) + _dot(er, c[2])
                     + bia[0:1], 0.0)
    uh = _dot(er, c[3]) + _dot(sh, c[4]) + _dot(so, c[5]) + bia[1:2]
    hh = jnp.maximum(_dot(eh, c[6]).reshape(bb, _H, 32) + uh[:, None, :], 0.0)
    uo = _dot(er, c[7]) + _dot(sh, c[8]) + _dot(so, c[9]) + bia[2:3]
    ho = jnp.maximum(_dot(eo, c[10]).reshape(bb, _O, 32) + uo[:, None, :], 0.0)

    sh2 = jnp.sum(hh, axis=1)
    so2 = jnp.sum(ho, axis=1)

    c2m = C2[...]
    hr2 = jnp.maximum(_dot(sh2, c2m[0]) + _dot(so2, c2m[1]) + _dot(hr, c2m[2])
                      + B2[...], 0.0)

    v = jnp.maximum(_dot(hr2, V1[...]) + c1[...], 0.0)
    v = jnp.maximum(_dot(v, V2[...]) + c2[...], 0.0)
    v = _dot(v, V3[...]) + c3[...]
    out_ref[...] = v


def _prep(params):
    def lin(layer):
        W, b = layer
        return W.T, b[None, :]

    rW1, rb1 = lin(params['w_r'][0])
    rW2, rb2 = lin(params['w_r'][1])
    hW1, hb1 = lin(params['w_h'][0])
    hW2, hb2 = lin(params['w_h'][1])
    oW1, ob1 = lin(params['w_o'][0])
    oW2, ob2 = lin(params['w_o'][1])

    def conv_combine(C):
        Ar_h = C['h2r']['W_rel'].T
        Ar_o = C['o2r']['W_rel'].T
        Ar_r = (C['h2r']['W_root'] + C['o2r']['W_root']).T
        br = C['h2r']['b_rel'] + C['o2r']['b_rel']
        Ah_r = C['r2h']['W_rel'].T
        Ah_sh = C['h2h']['W_rel'].T
        Ah_so = C['o2h']['W_rel'].T
        Ah_self = ((C['r2h']['W_root'] + C['o2h']['W_root']
                    + C['h2h']['W_root']).T - C['h2h']['W_rel'].T)
        bh = C['r2h']['b_rel'] + C['o2h']['b_rel'] + C['h2h']['b_rel']
        Ao_r = C['r2o']['W_rel'].T
        Ao_sh = C['h2o']['W_rel'].T
        Ao_so = C['o2o']['W_rel'].T
        Ao_self = ((C['r2o']['W_root'] + C['h2o']['W_root']
                    + C['o2o']['W_root']).T - C['o2o']['W_rel'].T)
        bo = C['r2o']['b_rel'] + C['h2o']['b_rel'] + C['o2o']['b_rel']
        mats = jnp.stack([Ar_h, Ar_o, Ar_r, Ah_r, Ah_sh, Ah_so, Ah_self,
                          Ao_r, Ao_sh, Ao_so, Ao_self])
        bias = jnp.stack([br, bh, bo])
        return mats, bias

    C1m, B1m = conv_combine(params['conv1'])
    C2all, B2all = conv_combine(params['conv2'])
    C2m = C2all[:3]
    B2m = B2all[0:1]

    V1, c1 = lin(params['value'][0])
    V2, c2 = lin(params['value'][1])
    V3, c3 = lin(params['value'][2])

    return (rW1, rb1, rW2, rb2, hW1, hb1, hW2, hb2, oW1, ob1, oW2, ob2,
            C1m, B1m, C2m, B2m, V1, c1, V2, c2, V3, c3)


@functools.partial(jax.jit, static_argnames=('interpret',))
def _run(state_input, weights, interpret=False):
    n_blocks = _BATCH // _BB

    def full(w):
        return pl.BlockSpec(w.shape, lambda i: (0,) * w.ndim)

    in_specs = [pl.BlockSpec((_BB, _H + _O, _IN), lambda i: (i, 0, 0))]
    in_specs += [full(w) for w in weights]
    out_spec = pl.BlockSpec((_BB, 1), lambda i: (i, 0))

    return pl.pallas_call(
        _fused_body,
        grid=(n_blocks,),
        in_specs=in_specs,
        out_specs=out_spec,
        out_shape=jax.ShapeDtypeStruct((_BATCH, 1), jnp.float32),
        interpret=interpret,
    )(state_input, *weights)


def kernel(state_input, params, dropout):
    weights = _prep(params)
    return _run(state_input, weights)
